# baseline (speedup 1.0000x reference)
import functools

import jax
import jax.numpy as jnp
from jax import lax
from jax.experimental import pallas as pl
from jax.experimental.pallas import tpu as pltpu
from jax.experimental.pallas import tpu_sc as plsc

D_MODEL = 1024
N_HEADS = 16
N_KV = 8
DH = 64
D_FF = 1024
N_EXP = 8
EPS = 1e-05
ROPE_BASE = 150000.0
ROPE_SCALE = 32.0
T = 2048
QKV_DIM = (N_HEADS + 2 * N_KV) * DH
NA = 2 * T
TB = 256
NT = NA // TB + (N_EXP - 1)
NP = NT * TB

F32 = jnp.float32
BF16 = jnp.bfloat16


def _qkv_body(x_ref, nw_ref, w_ref, b_ref, sin_ref, cos_ref,
              q_ref, k_ref, v_ref):
    x = x_ref[...]
    ms = jnp.mean(x * x, axis=1, keepdims=True)
    xn = (x * jax.lax.rsqrt(ms + EPS)) * nw_ref[...]
    qkv = jnp.dot(xn.astype(BF16), w_ref[...],
                  preferred_element_type=F32) + b_ref[...]
    sin = sin_ref[...]
    cos = cos_ref[...]

    def rope(h):
        h1 = h[:, :DH // 2]
        h2 = h[:, DH // 2:]
        return jnp.concatenate([h1 * cos - h2 * sin, h1 * sin + h2 * cos],
                               axis=1)

    tk = qkv.shape[0]
    for h in range(N_HEADS):
        q_ref[h] = (rope(qkv[:, h * DH:(h + 1) * DH]) * 0.125).astype(BF16)
    koff = N_HEADS * DH
    voff = koff + N_KV * DH
    ones = jnp.ones((tk, 1), F32)
    zeros = jnp.zeros((tk, DH - 1), F32)
    for h in range(N_KV):
        k_ref[h] = rope(qkv[:, koff + h * DH:koff + (h + 1) * DH]).astype(BF16)
        v_ref[h] = jnp.concatenate(
            [qkv[:, voff + h * DH:voff + (h + 1) * DH], ones, zeros],
            axis=1).astype(BF16)


TK1 = 512


def _qkv_call(x2d, nw, w_bf, b, sin, cos):
    return pl.pallas_call(
        _qkv_body,
        grid=(T // TK1,),
        in_specs=[
            pl.BlockSpec((TK1, D_MODEL), lambda i: (i, 0)),
            pl.BlockSpec((1, D_MODEL), lambda i: (0, 0)),
            pl.BlockSpec((D_MODEL, QKV_DIM), lambda i: (0, 0)),
            pl.BlockSpec((1, QKV_DIM), lambda i: (0, 0)),
            pl.BlockSpec((TK1, DH // 2), lambda i: (i, 0)),
            pl.BlockSpec((TK1, DH // 2), lambda i: (i, 0)),
        ],
        out_specs=(
            pl.BlockSpec((N_HEADS, TK1, DH), lambda i: (0, i, 0)),
            pl.BlockSpec((N_KV, TK1, DH), lambda i: (0, i, 0)),
            pl.BlockSpec((N_KV, TK1, 2 * DH), lambda i: (0, i, 0)),
        ),
        out_shape=(
            jax.ShapeDtypeStruct((N_HEADS, T, DH), BF16),
            jax.ShapeDtypeStruct((N_KV, T, DH), BF16),
            jax.ShapeDtypeStruct((N_KV, T, 2 * DH), BF16),
        ),
    )(x2d, nw, w_bf, b, sin, cos)


TQ = 512


KC = 512


def _attn_body(q_ref, k_ref, v_ref, sink_ref, x_ref, pw_ref, pb_ref,
               xa_ref, acc_ref):
    qi = pl.program_id(0)
    h = pl.program_id(1)
    q = q_ref[0]
    sink = sink_ref[0, 0, 0]

    def chunk(ki, carry, masked):
        m, l = carry
        kc = k_ref[0, pl.ds(ki * KC, KC), :]
        vc = v_ref[0, pl.ds(ki * KC, KC), :]
        lg = jax.lax.dot_general(q, kc, (((1,), (1,)), ((), ())),
                                 preferred_element_type=F32)
        if masked:
            row = jax.lax.broadcasted_iota(jnp.int32, (TQ, KC), 0)
            col = jax.lax.broadcasted_iota(jnp.int32, (TQ, KC), 1)
            lg = jnp.where(col > row, -jnp.inf, lg)
        mc = jnp.max(lg, axis=1, keepdims=True)
        mn = jnp.maximum(m, mc)
        scale = jnp.exp(m - mn)
        p = jnp.exp((lg - mn).astype(BF16))
        pv = jax.lax.dot_general(p, vc, (((1,), (0,)), ((), ())),
                                 preferred_element_type=F32)
        ln = l * scale + pv[:, DH:DH + 1]
        acc_ref[...] = acc_ref[...] * scale + pv[:, :DH]
        return mn, ln

    acc_ref[...] = jnp.zeros((TQ, DH), F32)
    m0 = jnp.full((TQ, 1), -jnp.inf, F32)
    l0 = jnp.zeros((TQ, 1), F32)
    carry = jax.lax.fori_loop(
        0, qi, lambda ki, c: chunk(ki, c, masked=False), (m0, l0))
    m, l = chunk(qi, carry, masked=True)
    mf = jnp.maximum(m, sink)
    lf = l * jnp.exp(m - mf) + jnp.exp(sink - mf)
    y = (acc_ref[...] * (jnp.exp(m - mf) / lf)).astype(BF16)
    part = jnp.dot(y, pw_ref[...], preferred_element_type=F32)

    @pl.when(h == 0)
    def _():
        xa_ref[...] = x_ref[...] + pb_ref[...] + part

    @pl.when(h > 0)
    def _():
        xa_ref[...] += part


def _attn_call(q, k, v, sink, x2d, pw_bf, pb):
    return pl.pallas_call(
        _attn_body,
        grid=(T // TQ, N_HEADS),
        in_specs=[
            pl.BlockSpec((1, TQ, DH), lambda i, h: (h, i, 0)),
            pl.BlockSpec((1, T, DH), lambda i, h: (h // 2, 0, 0)),
            pl.BlockSpec((1, T, 2 * DH), lambda i, h: (h // 2, 0, 0)),
            pl.BlockSpec((1, 1, 1), lambda i, h: (h, 0, 0)),
            pl.BlockSpec((TQ, D_MODEL), lambda i, h: (i, 0)),
            pl.BlockSpec((DH, D_MODEL), lambda i, h: (h, 0)),
            pl.BlockSpec((1, D_MODEL), lambda i, h: (0, 0)),
        ],
        out_specs=pl.BlockSpec((TQ, D_MODEL), lambda i, h: (i, 0)),
        out_shape=jax.ShapeDtypeStruct((T, D_MODEL), F32),
        scratch_shapes=[pltpu.VMEM((TQ, DH), F32)],
    )(q, k, v, sink, x2d, pw_bf, pb)


CH = 256


def _router_body(xa_ref, fw_ref, gw_ref, gb_ref,
                 x2_ref, w_ref, dest_ref, te_ref):
    xa = xa_ref[...]
    ms = jnp.mean(xa * xa, axis=1, keepdims=True)
    x2 = (xa * jax.lax.rsqrt(ms + EPS)) * fw_ref[...]
    x2_ref[...] = x2
    gl = jnp.dot(x2.astype(BF16), gw_ref[...],
                 preferred_element_type=F32) + gb_ref[...]
    lane = jax.lax.broadcasted_iota(jnp.int32, (T, 128), 1)
    gl = jnp.where(lane < N_EXP, gl, -jnp.inf)
    v1 = jnp.max(gl, axis=1, keepdims=True)
    i1 = jnp.min(jnp.where(gl == v1, lane, 128), axis=1, keepdims=True)
    gl2 = jnp.where(lane == i1, -jnp.inf, gl)
    v2 = jnp.max(gl2, axis=1, keepdims=True)
    i2 = jnp.min(jnp.where(gl2 == v2, lane, 128), axis=1, keepdims=True)
    w_ref[:, 0:1] = jax.nn.sigmoid(v1 - v2)
    w_ref[:, 1:2] = jax.nn.sigmoid(v2 - v1)

    o1 = (lane == i1).astype(F32)
    o2 = (lane == i2).astype(F32)
    occ = o1 + o2

    counts = jnp.sum(occ, axis=0, keepdims=True)
    padded = jnp.ceil(counts * (1.0 / TB)) * TB
    lr = jax.lax.broadcasted_iota(jnp.int32, (128, 128), 0)
    lc = jax.lax.broadcasted_iota(jnp.int32, (128, 128), 1)
    off = jnp.dot(padded, (lr < lc).astype(F32),
                  preferred_element_type=F32)

    r = jax.lax.broadcasted_iota(jnp.int32, (CH, CH), 0)
    c = jax.lax.broadcasted_iota(jnp.int32, (CH, CH), 1)
    tril = (c < r).astype(F32)
    carry = jnp.zeros((1, 128), F32)
    for ci in range(T // CH):
        blk = occ[ci * CH:(ci + 1) * CH, :]
        cum = jnp.dot(tril, blk, preferred_element_type=F32) + carry
        pos = cum + off
        o1b = o1[ci * CH:(ci + 1) * CH, :]
        o2b = o2[ci * CH:(ci + 1) * CH, :]
        d1 = jnp.sum(pos * o1b, axis=1, keepdims=True)
        d2 = jnp.sum(pos * o2b, axis=1, keepdims=True)
        dest_ref[ci * CH:(ci + 1) * CH, 0:1] = d1.astype(jnp.int32)
        dest_ref[ci * CH:(ci + 1) * CH, 1:2] = d2.astype(jnp.int32)
        carry = carry + jnp.sum(blk, axis=0, keepdims=True)

    tiles = padded * (1.0 / TB)
    cumtiles = jnp.dot(tiles, (lr <= lc).astype(F32),
                       preferred_element_type=F32)
    jl = jax.lax.broadcasted_iota(jnp.int32, (1, 128), 1).astype(F32)
    te = jnp.zeros((1, 128), F32)
    for e in range(N_EXP - 1):
        ce = jnp.sum(jnp.where(lane[0:1, :] == e, cumtiles, 0.0))
        te = te + (jl >= ce).astype(F32)
    te_ref[...] = te.astype(jnp.int32)


def _router_call(xa, fw, gw_bf, gb):
    return pl.pallas_call(
        _router_body,
        out_shape=(
            jax.ShapeDtypeStruct((T, D_MODEL), F32),
            jax.ShapeDtypeStruct((T, 2), F32),
            jax.ShapeDtypeStruct((T, 2), jnp.int32),
            jax.ShapeDtypeStruct((1, 128), jnp.int32),
        ),
    )(xa, fw, gw_bf, gb)


_SC_CHUNK = 64
_SC_CORES = 2
_SC_SUBCORES = 16


def _make_sc_dispatch():
    nw = _SC_CORES * _SC_SUBCORES
    a_per_w = NA // nw
    mesh = plsc.VectorSubcoreMesh(core_axis_name="c", subcore_axis_name="s")

    @functools.partial(
        pl.kernel, mesh=mesh,
        out_type=jax.ShapeDtypeStruct((NP, D_MODEL), F32),
        scratch_types=[
            pltpu.VMEM((_SC_CHUNK,), jnp.int32),
            pltpu.VMEM((_SC_CHUNK,), jnp.int32),
            pltpu.VMEM((_SC_CHUNK, D_MODEL), F32),
            pltpu.SemaphoreType.DMA,
        ],
    )
    def dispatch(x2_hbm, tok_hbm, dest_hbm, xs_hbm, tok_v, dest_v, rows_v,
                 sem):
        wid = lax.axis_index("s") * _SC_CORES + lax.axis_index("c")
        for ci in range(a_per_w // _SC_CHUNK):
            base = wid * a_per_w + ci * _SC_CHUNK
            pltpu.sync_copy(tok_hbm.at[pl.ds(base, _SC_CHUNK)], tok_v)
            pltpu.sync_copy(dest_hbm.at[pl.ds(base, _SC_CHUNK)], dest_v)
            pltpu.async_copy(x2_hbm.at[tok_v], rows_v, sem).wait()
            pltpu.async_copy(rows_v, xs_hbm.at[dest_v], sem).wait()

    return dispatch


def _make_sc_combine():
    nw = _SC_CORES * _SC_SUBCORES
    a_per_w = NA // nw
    mesh = plsc.VectorSubcoreMesh(core_axis_name="c", subcore_axis_name="s")

    @functools.partial(
        pl.kernel, mesh=mesh,
        out_type=jax.ShapeDtypeStruct((NA, D_MODEL), F32),
        scratch_types=[
            pltpu.VMEM((_SC_CHUNK,), jnp.int32),
            pltpu.VMEM((_SC_CHUNK, D_MODEL), F32),
            pltpu.SemaphoreType.DMA,
        ],
    )
    def combine(ys_hbm, dest_hbm, yg_hbm, dest_v, rows_v, sem):
        wid = lax.axis_index("s") * _SC_CORES + lax.axis_index("c")
        for ci in range(a_per_w // _SC_CHUNK):
            base = wid * a_per_w + ci * _SC_CHUNK
            pltpu.sync_copy(dest_hbm.at[pl.ds(base, _SC_CHUNK)], dest_v)
            pltpu.async_copy(ys_hbm.at[dest_v], rows_v, sem).wait()
            pltpu.sync_copy(rows_v, yg_hbm.at[pl.ds(base, _SC_CHUNK)])

    return combine


def _ffn_body(te_ref, xs_ref, up_ref, dn_ref, ys_ref):
    xb = xs_ref[...].astype(BF16)
    h = jnp.dot(xb, up_ref[0].astype(BF16), preferred_element_type=F32)
    u = h[:, :D_FF]
    g = h[:, D_FF:]
    act = (jax.nn.silu(g) * u).astype(BF16)
    ys_ref[...] = jnp.dot(act, dn_ref[0].astype(BF16),
                          preferred_element_type=F32)


def _ffn_call(te, xs, up_bf, dn_bf):
    grid_spec = pltpu.PrefetchScalarGridSpec(
        num_scalar_prefetch=1,
        grid=(NT,),
        in_specs=[
            pl.BlockSpec((TB, D_MODEL), lambda i, te: (i, 0)),
            pl.BlockSpec((1, D_MODEL, 2 * D_FF), lambda i, te: (te[i], 0, 0)),
            pl.BlockSpec((1, D_FF, D_MODEL), lambda i, te: (te[i], 0, 0)),
        ],
        out_specs=pl.BlockSpec((TB, D_MODEL), lambda i, te: (i, 0)),
    )
    return pl.pallas_call(
        _ffn_body, grid_spec=grid_spec,
        out_shape=jax.ShapeDtypeStruct((NP, D_MODEL), F32),
    )(te, xs, up_bf, dn_bf)


TC2 = 512


def _combine_body(xa_ref, yg_ref, w_ref, out_ref):
    w = w_ref[...]
    out_ref[...] = (xa_ref[...] + w[:, 0:1] * yg_ref[:, 0, :]
                    + w[:, 1:2] * yg_ref[:, 1, :])


def _combine_call(xa, yg3, w):
    return pl.pallas_call(
        _combine_body,
        grid=(T // TC2,),
        in_specs=[
            pl.BlockSpec((TC2, D_MODEL), lambda i: (i, 0)),
            pl.BlockSpec((TC2, 2, D_MODEL), lambda i: (i, 0, 0)),
            pl.BlockSpec((TC2, 2), lambda i: (i, 0)),
        ],
        out_specs=pl.BlockSpec((TC2, D_MODEL), lambda i: (i, 0)),
        out_shape=jax.ShapeDtypeStruct((T, D_MODEL), F32),
    )(xa, yg3, w)


def _rope_tables():
    pos = jnp.arange(T, dtype=F32) / ROPE_SCALE
    idx = jnp.arange(0, DH, 2, dtype=F32)
    inv_freq = 1.0 / (ROPE_BASE ** (idx / DH))
    freqs = jnp.einsum('t,f->tf', pos, inv_freq)
    return jnp.sin(freqs), jnp.cos(freqs)


def _deinterleave_qkv_w(qkv_w, qkv_b):
    nr = N_HEADS + N_KV
    roped = qkv_w[:, :nr * DH].reshape(D_MODEL, nr, DH // 2, 2)
    roped = roped.transpose(0, 1, 3, 2).reshape(D_MODEL, nr * DH)
    w = jnp.concatenate([roped, qkv_w[:, nr * DH:]], axis=1)
    br = qkv_b[:nr * DH].reshape(nr, DH // 2, 2)
    br = br.transpose(0, 2, 1).reshape(nr * DH)
    b = jnp.concatenate([br, qkv_b[nr * DH:]])
    return w, b


def kernel(x, attn_norm_w, ffn_norm_w, qkv_w, qkv_b, proj_w, proj_b,
           attn_sink, gate_w, gate_b, up_w, down_w):
    B = x.shape[0]
    x2d = x.reshape(T, D_MODEL)
    sin, cos = _rope_tables()
    w_perm, b_perm = _deinterleave_qkv_w(qkv_w, qkv_b)

    q, k, v = _qkv_call(
        x2d, attn_norm_w.reshape(1, D_MODEL), w_perm.astype(BF16),
        b_perm.reshape(1, QKV_DIM), sin, cos)

    xa = _attn_call(q, k, v, attn_sink.reshape(N_HEADS, 1, 1), x2d,
                    proj_w.astype(BF16), proj_b.reshape(1, D_MODEL))

    gw_pad = jnp.zeros((D_MODEL, 128), F32).at[:, :N_EXP].set(gate_w)
    gb_pad = jnp.zeros((1, 128), F32).at[0, :N_EXP].set(gate_b)
    x2, w12, dest, te = _router_call(
        xa, ffn_norm_w.reshape(1, D_MODEL), gw_pad.astype(BF16), gb_pad)

    dest_flat = dest.reshape(NA)
    tok = jnp.arange(NA, dtype=jnp.int32) // 2

    xs = _make_sc_dispatch()(x2, tok, dest_flat)
    ys = _ffn_call(te.reshape(128)[:NT], xs, up_w, down_w)
    yg = _make_sc_combine()(ys, dest_flat)

    out = _combine_call(xa, yg.reshape(T, 2, D_MODEL), w12)
    return out.reshape(B, T, D_MODEL)


# speedup vs baseline: 1.1096x; 1.1096x over previous
import functools

import jax
import jax.numpy as jnp
from jax import lax
from jax.experimental import pallas as pl
from jax.experimental.pallas import tpu as pltpu
from jax.experimental.pallas import tpu_sc as plsc

D_MODEL = 1024
N_HEADS = 16
N_KV = 8
DH = 64
D_FF = 1024
N_EXP = 8
EPS = 1e-05
ROPE_BASE = 150000.0
ROPE_SCALE = 32.0
T = 2048
QKV_DIM = (N_HEADS + 2 * N_KV) * DH
NA = 2 * T
TB = 256
NT = NA // TB + (N_EXP - 1)
NP = NT * TB

F32 = jnp.float32
BF16 = jnp.bfloat16


def _qkv_body(x_ref, nw_ref, w_ref, b_ref, sin_ref, cos_ref,
              q_ref, k_ref, v_ref):
    x = x_ref[...]
    ms = jnp.mean(x * x, axis=1, keepdims=True)
    xn = (x * jax.lax.rsqrt(ms + EPS)) * nw_ref[...]
    qkv = jnp.dot(xn.astype(BF16), w_ref[...],
                  preferred_element_type=F32) + b_ref[...]
    sin = sin_ref[...]
    cos = cos_ref[...]

    def rope(h):
        h1 = h[:, :DH // 2]
        h2 = h[:, DH // 2:]
        return jnp.concatenate([h1 * cos - h2 * sin, h1 * sin + h2 * cos],
                               axis=1)

    tk = qkv.shape[0]
    for h in range(N_HEADS):
        q_ref[h] = (rope(qkv[:, h * DH:(h + 1) * DH]) * 0.125).astype(BF16)
    koff = N_HEADS * DH
    voff = koff + N_KV * DH
    ones = jnp.ones((tk, 1), F32)
    zeros = jnp.zeros((tk, DH - 1), F32)
    for h in range(N_KV):
        k_ref[h] = rope(qkv[:, koff + h * DH:koff + (h + 1) * DH]).astype(BF16)
        v_ref[h] = jnp.concatenate(
            [qkv[:, voff + h * DH:voff + (h + 1) * DH], ones, zeros],
            axis=1).astype(BF16)


TK1 = 512


def _qkv_call(x2d, nw, w_bf, b, sin, cos):
    return pl.pallas_call(
        _qkv_body,
        grid=(T // TK1,),
        in_specs=[
            pl.BlockSpec((TK1, D_MODEL), lambda i: (i, 0)),
            pl.BlockSpec((1, D_MODEL), lambda i: (0, 0)),
            pl.BlockSpec((D_MODEL, QKV_DIM), lambda i: (0, 0)),
            pl.BlockSpec((1, QKV_DIM), lambda i: (0, 0)),
            pl.BlockSpec((TK1, DH // 2), lambda i: (i, 0)),
            pl.BlockSpec((TK1, DH // 2), lambda i: (i, 0)),
        ],
        out_specs=(
            pl.BlockSpec((N_HEADS, TK1, DH), lambda i: (0, i, 0)),
            pl.BlockSpec((N_KV, TK1, DH), lambda i: (0, i, 0)),
            pl.BlockSpec((N_KV, TK1, 2 * DH), lambda i: (0, i, 0)),
        ),
        out_shape=(
            jax.ShapeDtypeStruct((N_HEADS, T, DH), BF16),
            jax.ShapeDtypeStruct((N_KV, T, DH), BF16),
            jax.ShapeDtypeStruct((N_KV, T, 2 * DH), BF16),
        ),
    )(x2d, nw, w_bf, b, sin, cos)


TQ = 512


KC = 512


def _attn_body(q_ref, k_ref, v_ref, sink_ref, x_ref, pw_ref, pb_ref,
               xa_ref, acc_ref):
    qi = pl.program_id(0)
    h = pl.program_id(1)
    q = q_ref[0]
    sink = sink_ref[0, 0, 0]

    def chunk(ki, m, masked):
        kc = k_ref[0, pl.ds(ki * KC, KC), :]
        vc = v_ref[0, pl.ds(ki * KC, KC), :]
        lg = jax.lax.dot_general(q, kc, (((1,), (1,)), ((), ())),
                                 preferred_element_type=F32)
        if masked:
            row = jax.lax.broadcasted_iota(jnp.int32, (TQ, KC), 0)
            col = jax.lax.broadcasted_iota(jnp.int32, (TQ, KC), 1)
            lg = jnp.where(col > row, -jnp.inf, lg)
        mc = jnp.max(lg, axis=1, keepdims=True)
        mn = jnp.maximum(m, mc)
        p = jnp.exp((lg - mn).astype(BF16))
        pv = jax.lax.dot_general(p, vc, (((1,), (0,)), ((), ())),
                                 preferred_element_type=F32)
        acc_ref[...] = acc_ref[...] * jnp.exp(m - mn) + pv
        return mn

    acc_ref[...] = jnp.zeros((TQ, 2 * DH), F32)
    m0 = jnp.full((TQ, 1), -jnp.inf, F32)
    m = jax.lax.fori_loop(
        0, qi, lambda ki, c: chunk(ki, c, masked=False), m0)
    m = chunk(qi, m, masked=True)
    mf = jnp.maximum(m, sink)
    acc = acc_ref[...]
    lf = acc[:, DH:DH + 1] * jnp.exp(m - mf) + jnp.exp(sink - mf)
    y = (acc[:, :DH] * (jnp.exp(m - mf) / lf)).astype(BF16)
    part = jnp.dot(y, pw_ref[...], preferred_element_type=F32)

    @pl.when(h == 0)
    def _():
        xa_ref[...] = x_ref[...] + pb_ref[...] + part

    @pl.when(h > 0)
    def _():
        xa_ref[...] += part


def _attn_call(q, k, v, sink, x2d, pw_bf, pb):
    return pl.pallas_call(
        _attn_body,
        grid=(T // TQ, N_HEADS),
        in_specs=[
            pl.BlockSpec((1, TQ, DH), lambda i, h: (h, i, 0)),
            pl.BlockSpec((1, T, DH), lambda i, h: (h // 2, 0, 0)),
            pl.BlockSpec((1, T, 2 * DH), lambda i, h: (h // 2, 0, 0)),
            pl.BlockSpec((1, 1, 1), lambda i, h: (h, 0, 0)),
            pl.BlockSpec((TQ, D_MODEL), lambda i, h: (i, 0)),
            pl.BlockSpec((DH, D_MODEL), lambda i, h: (h, 0)),
            pl.BlockSpec((1, D_MODEL), lambda i, h: (0, 0)),
        ],
        out_specs=pl.BlockSpec((TQ, D_MODEL), lambda i, h: (i, 0)),
        out_shape=jax.ShapeDtypeStruct((T, D_MODEL), F32),
        scratch_shapes=[pltpu.VMEM((TQ, 2 * DH), F32)],
    )(q, k, v, sink, x2d, pw_bf, pb)


CH = 256


def _router_body(xa_ref, fw_ref, gw_ref, gb_ref,
                 x2_ref, w_ref, dest_ref, te_ref):
    xa = xa_ref[...]
    ms = jnp.mean(xa * xa, axis=1, keepdims=True)
    x2 = (xa * jax.lax.rsqrt(ms + EPS)) * fw_ref[...]
    x2_ref[...] = x2
    gl = jnp.dot(x2.astype(BF16), gw_ref[...],
                 preferred_element_type=F32) + gb_ref[...]
    lane = jax.lax.broadcasted_iota(jnp.int32, (T, 128), 1)
    gl = jnp.where(lane < N_EXP, gl, -jnp.inf)
    v1 = jnp.max(gl, axis=1, keepdims=True)
    i1 = jnp.min(jnp.where(gl == v1, lane, 128), axis=1, keepdims=True)
    gl2 = jnp.where(lane == i1, -jnp.inf, gl)
    v2 = jnp.max(gl2, axis=1, keepdims=True)
    i2 = jnp.min(jnp.where(gl2 == v2, lane, 128), axis=1, keepdims=True)
    w_ref[:, 0:1] = jax.nn.sigmoid(v1 - v2)
    w_ref[:, 1:2] = jax.nn.sigmoid(v2 - v1)

    o1 = (lane == i1).astype(F32)
    o2 = (lane == i2).astype(F32)
    occ = o1 + o2

    counts = jnp.sum(occ, axis=0, keepdims=True)
    padded = jnp.ceil(counts * (1.0 / TB)) * TB
    lr = jax.lax.broadcasted_iota(jnp.int32, (128, 128), 0)
    lc = jax.lax.broadcasted_iota(jnp.int32, (128, 128), 1)
    off = jnp.dot(padded, (lr < lc).astype(F32),
                  preferred_element_type=F32)

    r = jax.lax.broadcasted_iota(jnp.int32, (CH, CH), 0)
    c = jax.lax.broadcasted_iota(jnp.int32, (CH, CH), 1)
    tril = (c < r).astype(F32)
    carry = jnp.zeros((1, 128), F32)
    for ci in range(T // CH):
        blk = occ[ci * CH:(ci + 1) * CH, :]
        cum = jnp.dot(tril, blk, preferred_element_type=F32) + carry
        pos = cum + off
        o1b = o1[ci * CH:(ci + 1) * CH, :]
        o2b = o2[ci * CH:(ci + 1) * CH, :]
        d1 = jnp.sum(pos * o1b, axis=1, keepdims=True)
        d2 = jnp.sum(pos * o2b, axis=1, keepdims=True)
        dest_ref[ci * CH:(ci + 1) * CH, 0:1] = d1.astype(jnp.int32)
        dest_ref[ci * CH:(ci + 1) * CH, 1:2] = d2.astype(jnp.int32)
        carry = carry + jnp.sum(blk, axis=0, keepdims=True)

    tiles = padded * (1.0 / TB)
    cumtiles = jnp.dot(tiles, (lr <= lc).astype(F32),
                       preferred_element_type=F32)
    jl = jax.lax.broadcasted_iota(jnp.int32, (1, 128), 1).astype(F32)
    te = jnp.zeros((1, 128), F32)
    for e in range(N_EXP - 1):
        ce = jnp.sum(jnp.where(lane[0:1, :] == e, cumtiles, 0.0))
        te = te + (jl >= ce).astype(F32)
    te_ref[...] = te.astype(jnp.int32)


def _router_call(xa, fw, gw_bf, gb):
    return pl.pallas_call(
        _router_body,
        out_shape=(
            jax.ShapeDtypeStruct((T, D_MODEL), F32),
            jax.ShapeDtypeStruct((T, 2), F32),
            jax.ShapeDtypeStruct((T, 2), jnp.int32),
            jax.ShapeDtypeStruct((1, 128), jnp.int32),
        ),
    )(xa, fw, gw_bf, gb)


_SC_CHUNK = 64
_SC_CORES = 2
_SC_SUBCORES = 16


def _make_sc_dispatch():
    nw = _SC_CORES * _SC_SUBCORES
    a_per_w = NA // nw
    mesh = plsc.VectorSubcoreMesh(core_axis_name="c", subcore_axis_name="s")

    @functools.partial(
        pl.kernel, mesh=mesh,
        out_type=jax.ShapeDtypeStruct((NP, D_MODEL), F32),
        scratch_types=[
            pltpu.VMEM((_SC_CHUNK,), jnp.int32),
            pltpu.VMEM((_SC_CHUNK,), jnp.int32),
            pltpu.VMEM((_SC_CHUNK, D_MODEL), F32),
            pltpu.SemaphoreType.DMA,
        ],
    )
    def dispatch(x2_hbm, tok_hbm, dest_hbm, xs_hbm, tok_v, dest_v, rows_v,
                 sem):
        wid = lax.axis_index("s") * _SC_CORES + lax.axis_index("c")
        for ci in range(a_per_w // _SC_CHUNK):
            base = wid * a_per_w + ci * _SC_CHUNK
            pltpu.sync_copy(tok_hbm.at[pl.ds(base, _SC_CHUNK)], tok_v)
            pltpu.sync_copy(dest_hbm.at[pl.ds(base, _SC_CHUNK)], dest_v)
            pltpu.async_copy(x2_hbm.at[tok_v], rows_v, sem).wait()
            pltpu.async_copy(rows_v, xs_hbm.at[dest_v], sem).wait()

    return dispatch


def _make_sc_combine():
    nw = _SC_CORES * _SC_SUBCORES
    a_per_w = NA // nw
    mesh = plsc.VectorSubcoreMesh(core_axis_name="c", subcore_axis_name="s")

    @functools.partial(
        pl.kernel, mesh=mesh,
        out_type=jax.ShapeDtypeStruct((NA, D_MODEL), F32),
        scratch_types=[
            pltpu.VMEM((_SC_CHUNK,), jnp.int32),
            pltpu.VMEM((_SC_CHUNK, D_MODEL), F32),
            pltpu.SemaphoreType.DMA,
        ],
    )
    def combine(ys_hbm, dest_hbm, yg_hbm, dest_v, rows_v, sem):
        wid = lax.axis_index("s") * _SC_CORES + lax.axis_index("c")
        for ci in range(a_per_w // _SC_CHUNK):
            base = wid * a_per_w + ci * _SC_CHUNK
            pltpu.sync_copy(dest_hbm.at[pl.ds(base, _SC_CHUNK)], dest_v)
            pltpu.async_copy(ys_hbm.at[dest_v], rows_v, sem).wait()
            pltpu.sync_copy(rows_v, yg_hbm.at[pl.ds(base, _SC_CHUNK)])

    return combine


def _ffn_body(te_ref, xs_ref, up_ref, dn_ref, ys_ref):
    xb = xs_ref[...].astype(BF16)
    h = jnp.dot(xb, up_ref[0].astype(BF16), preferred_element_type=F32)
    u = h[:, :D_FF]
    g = h[:, D_FF:]
    act = (jax.nn.silu(g) * u).astype(BF16)
    ys_ref[...] = jnp.dot(act, dn_ref[0].astype(BF16),
                          preferred_element_type=F32)


def _ffn_call(te, xs, up_bf, dn_bf):
    grid_spec = pltpu.PrefetchScalarGridSpec(
        num_scalar_prefetch=1,
        grid=(NT,),
        in_specs=[
            pl.BlockSpec((TB, D_MODEL), lambda i, te: (i, 0)),
            pl.BlockSpec((1, D_MODEL, 2 * D_FF), lambda i, te: (te[i], 0, 0)),
            pl.BlockSpec((1, D_FF, D_MODEL), lambda i, te: (te[i], 0, 0)),
        ],
        out_specs=pl.BlockSpec((TB, D_MODEL), lambda i, te: (i, 0)),
    )
    return pl.pallas_call(
        _ffn_body, grid_spec=grid_spec,
        out_shape=jax.ShapeDtypeStruct((NP, D_MODEL), F32),
    )(te, xs, up_bf, dn_bf)


TC2 = 512


def _combine_body(xa_ref, yg_ref, w_ref, out_ref):
    w = w_ref[...]
    out_ref[...] = (xa_ref[...] + w[:, 0:1] * yg_ref[:, 0, :]
                    + w[:, 1:2] * yg_ref[:, 1, :])


def _combine_call(xa, yg3, w):
    return pl.pallas_call(
        _combine_body,
        grid=(T // TC2,),
        in_specs=[
            pl.BlockSpec((TC2, D_MODEL), lambda i: (i, 0)),
            pl.BlockSpec((TC2, 2, D_MODEL), lambda i: (i, 0, 0)),
            pl.BlockSpec((TC2, 2), lambda i: (i, 0)),
        ],
        out_specs=pl.BlockSpec((TC2, D_MODEL), lambda i: (i, 0)),
        out_shape=jax.ShapeDtypeStruct((T, D_MODEL), F32),
    )(xa, yg3, w)


def _rope_tables():
    pos = jnp.arange(T, dtype=F32) / ROPE_SCALE
    idx = jnp.arange(0, DH, 2, dtype=F32)
    inv_freq = 1.0 / (ROPE_BASE ** (idx / DH))
    freqs = jnp.einsum('t,f->tf', pos, inv_freq)
    return jnp.sin(freqs), jnp.cos(freqs)


def _deinterleave_qkv_w(qkv_w, qkv_b):
    nr = N_HEADS + N_KV
    roped = qkv_w[:, :nr * DH].reshape(D_MODEL, nr, DH // 2, 2)
    roped = roped.transpose(0, 1, 3, 2).reshape(D_MODEL, nr * DH)
    w = jnp.concatenate([roped, qkv_w[:, nr * DH:]], axis=1)
    br = qkv_b[:nr * DH].reshape(nr, DH // 2, 2)
    br = br.transpose(0, 2, 1).reshape(nr * DH)
    b = jnp.concatenate([br, qkv_b[nr * DH:]])
    return w, b


def kernel(x, attn_norm_w, ffn_norm_w, qkv_w, qkv_b, proj_w, proj_b,
           attn_sink, gate_w, gate_b, up_w, down_w):
    B = x.shape[0]
    x2d = x.reshape(T, D_MODEL)
    sin, cos = _rope_tables()
    w_perm, b_perm = _deinterleave_qkv_w(qkv_w, qkv_b)

    q, k, v = _qkv_call(
        x2d, attn_norm_w.reshape(1, D_MODEL), w_perm.astype(BF16),
        b_perm.reshape(1, QKV_DIM), sin, cos)

    xa = _attn_call(q, k, v, attn_sink.reshape(N_HEADS, 1, 1), x2d,
                    proj_w.astype(BF16), proj_b.reshape(1, D_MODEL))

    gw_pad = jnp.zeros((D_MODEL, 128), F32).at[:, :N_EXP].set(gate_w)
    gb_pad = jnp.zeros((1, 128), F32).at[0, :N_EXP].set(gate_b)
    x2, w12, dest, te = _router_call(
        xa, ffn_norm_w.reshape(1, D_MODEL), gw_pad.astype(BF16), gb_pad)

    dest_flat = dest.reshape(NA)
    tok = jnp.arange(NA, dtype=jnp.int32) // 2

    xs = _make_sc_dispatch()(x2, tok, dest_flat)
    ys = _ffn_call(te.reshape(128)[:NT], xs, up_w, down_w)
    yg = _make_sc_combine()(ys, dest_flat)

    out = _combine_call(xa, yg.reshape(T, 2, D_MODEL), w12)
    return out.reshape(B, T, D_MODEL)


# speedup vs baseline: 1.1333x; 1.0214x over previous
import functools

import jax
import jax.numpy as jnp
from jax import lax
from jax.experimental import pallas as pl
from jax.experimental.pallas import tpu as pltpu
from jax.experimental.pallas import tpu_sc as plsc

D_MODEL = 1024
N_HEADS = 16
N_KV = 8
DH = 64
D_FF = 1024
N_EXP = 8
EPS = 1e-05
ROPE_BASE = 150000.0
ROPE_SCALE = 32.0
T = 2048
QKV_DIM = (N_HEADS + 2 * N_KV) * DH
NA = 2 * T
TB = 256
NT = NA // TB + (N_EXP - 1)
NP = NT * TB

F32 = jnp.float32
BF16 = jnp.bfloat16


def _qkv_body(x_ref, nw_ref, w_ref, b_ref, sin_ref, cos_ref,
              q_ref, k_ref, v_ref):
    x = x_ref[...]
    ms = jnp.mean(x * x, axis=1, keepdims=True)
    xn = (x * jax.lax.rsqrt(ms + EPS)) * nw_ref[...]
    qkv = jnp.dot(xn.astype(BF16), w_ref[...],
                  preferred_element_type=F32) + b_ref[...]
    sin = sin_ref[...]
    cos = cos_ref[...]
    tk0 = sin.shape[0]
    even = jax.lax.broadcasted_iota(jnp.int32, (tk0, DH), 1) % 2 == 0

    def rope(h):
        swap = jnp.where(even, pltpu.roll(h, DH - 1, axis=1),
                         pltpu.roll(h, 1, axis=1))
        return h * cos + swap * sin

    tk = qkv.shape[0]
    for h in range(N_HEADS):
        q_ref[h] = (rope(qkv[:, h * DH:(h + 1) * DH]) * 0.125).astype(BF16)
    koff = N_HEADS * DH
    voff = koff + N_KV * DH
    ones = jnp.ones((tk, 1), F32)
    zeros = jnp.zeros((tk, DH - 1), F32)
    for h in range(N_KV):
        k_ref[h] = rope(qkv[:, koff + h * DH:koff + (h + 1) * DH]).astype(BF16)
        v_ref[h] = jnp.concatenate(
            [qkv[:, voff + h * DH:voff + (h + 1) * DH], ones, zeros],
            axis=1).astype(BF16)


TK1 = 512


def _qkv_call(x2d, nw, w_bf, b, sin, cos):
    return pl.pallas_call(
        _qkv_body,
        grid=(T // TK1,),
        in_specs=[
            pl.BlockSpec((TK1, D_MODEL), lambda i: (i, 0)),
            pl.BlockSpec((1, D_MODEL), lambda i: (0, 0)),
            pl.BlockSpec((D_MODEL, QKV_DIM), lambda i: (0, 0)),
            pl.BlockSpec((1, QKV_DIM), lambda i: (0, 0)),
            pl.BlockSpec((TK1, DH), lambda i: (i, 0)),
            pl.BlockSpec((TK1, DH), lambda i: (i, 0)),
        ],
        out_specs=(
            pl.BlockSpec((N_HEADS, TK1, DH), lambda i: (0, i, 0)),
            pl.BlockSpec((N_KV, TK1, DH), lambda i: (0, i, 0)),
            pl.BlockSpec((N_KV, TK1, 2 * DH), lambda i: (0, i, 0)),
        ),
        out_shape=(
            jax.ShapeDtypeStruct((N_HEADS, T, DH), BF16),
            jax.ShapeDtypeStruct((N_KV, T, DH), BF16),
            jax.ShapeDtypeStruct((N_KV, T, 2 * DH), BF16),
        ),
    )(x2d, nw, w_bf, b, sin, cos)


TQ = 512


KC = 512


def _attn_body(q_ref, k_ref, v_ref, sink_ref, x_ref, pw_ref, pb_ref,
               xa_ref, acc_ref):
    qi = pl.program_id(0)
    h = pl.program_id(1)
    q = q_ref[0]
    sink = sink_ref[0, 0, 0]

    def chunk(ki, m, masked):
        kc = k_ref[0, pl.ds(ki * KC, KC), :]
        vc = v_ref[0, pl.ds(ki * KC, KC), :]
        lg = jax.lax.dot_general(q, kc, (((1,), (1,)), ((), ())),
                                 preferred_element_type=F32)
        if masked:
            row = jax.lax.broadcasted_iota(jnp.int32, (TQ, KC), 0)
            col = jax.lax.broadcasted_iota(jnp.int32, (TQ, KC), 1)
            lg = jnp.where(col > row, -jnp.inf, lg)
        mc = jnp.max(lg, axis=1, keepdims=True)
        mn = jnp.maximum(m, mc)
        p = jnp.exp((lg - mn).astype(BF16))
        pv = jax.lax.dot_general(p, vc, (((1,), (0,)), ((), ())),
                                 preferred_element_type=F32)
        acc_ref[...] = acc_ref[...] * jnp.exp(m - mn) + pv
        return mn

    acc_ref[...] = jnp.zeros((TQ, 2 * DH), F32)
    m0 = jnp.full((TQ, 1), -jnp.inf, F32)
    m = jax.lax.fori_loop(
        0, qi, lambda ki, c: chunk(ki, c, masked=False), m0)
    m = chunk(qi, m, masked=True)
    mf = jnp.maximum(m, sink)
    acc = acc_ref[...]
    lf = acc[:, DH:DH + 1] * jnp.exp(m - mf) + jnp.exp(sink - mf)
    y = (acc[:, :DH] * (jnp.exp(m - mf) / lf)).astype(BF16)
    part = jnp.dot(y, pw_ref[...], preferred_element_type=F32)

    @pl.when(h == 0)
    def _():
        xa_ref[...] = x_ref[...] + pb_ref[...] + part

    @pl.when(h > 0)
    def _():
        xa_ref[...] += part


def _attn_call(q, k, v, sink, x2d, pw_bf, pb):
    return pl.pallas_call(
        _attn_body,
        grid=(T // TQ, N_HEADS),
        in_specs=[
            pl.BlockSpec((1, TQ, DH), lambda i, h: (h, i, 0)),
            pl.BlockSpec((1, T, DH), lambda i, h: (h // 2, 0, 0)),
            pl.BlockSpec((1, T, 2 * DH), lambda i, h: (h // 2, 0, 0)),
            pl.BlockSpec((1, 1, 1), lambda i, h: (h, 0, 0)),
            pl.BlockSpec((TQ, D_MODEL), lambda i, h: (i, 0)),
            pl.BlockSpec((DH, D_MODEL), lambda i, h: (h, 0)),
            pl.BlockSpec((1, D_MODEL), lambda i, h: (0, 0)),
        ],
        out_specs=pl.BlockSpec((TQ, D_MODEL), lambda i, h: (i, 0)),
        out_shape=jax.ShapeDtypeStruct((T, D_MODEL), F32),
        scratch_shapes=[pltpu.VMEM((TQ, 2 * DH), F32)],
    )(q, k, v, sink, x2d, pw_bf, pb)


CH = 256


def _router_body(xa_ref, fw_ref, gw_ref, gb_ref,
                 x2_ref, w_ref, dest_ref, te_ref):
    xa = xa_ref[...]
    ms = jnp.mean(xa * xa, axis=1, keepdims=True)
    x2 = (xa * jax.lax.rsqrt(ms + EPS)) * fw_ref[...]
    x2_ref[...] = x2
    gl = jnp.dot(x2.astype(BF16), gw_ref[...],
                 preferred_element_type=F32) + gb_ref[...]
    lane = jax.lax.broadcasted_iota(jnp.int32, (T, 128), 1)
    gl = jnp.where(lane < N_EXP, gl, -jnp.inf)
    v1 = jnp.max(gl, axis=1, keepdims=True)
    i1 = jnp.min(jnp.where(gl == v1, lane, 128), axis=1, keepdims=True)
    gl2 = jnp.where(lane == i1, -jnp.inf, gl)
    v2 = jnp.max(gl2, axis=1, keepdims=True)
    i2 = jnp.min(jnp.where(gl2 == v2, lane, 128), axis=1, keepdims=True)
    w_ref[:, 0:1] = jax.nn.sigmoid(v1 - v2)
    w_ref[:, 1:2] = jax.nn.sigmoid(v2 - v1)

    o1 = (lane == i1).astype(F32)
    o2 = (lane == i2).astype(F32)
    occ = o1 + o2

    counts = jnp.sum(occ, axis=0, keepdims=True)
    padded = jnp.ceil(counts * (1.0 / TB)) * TB
    lr = jax.lax.broadcasted_iota(jnp.int32, (128, 128), 0)
    lc = jax.lax.broadcasted_iota(jnp.int32, (128, 128), 1)
    off = jnp.dot(padded, (lr < lc).astype(F32),
                  preferred_element_type=F32)

    r = jax.lax.broadcasted_iota(jnp.int32, (CH, CH), 0)
    c = jax.lax.broadcasted_iota(jnp.int32, (CH, CH), 1)
    tril = (c < r).astype(F32)
    carry = jnp.zeros((1, 128), F32)
    for ci in range(T // CH):
        blk = occ[ci * CH:(ci + 1) * CH, :]
        cum = jnp.dot(tril, blk, preferred_element_type=F32) + carry
        pos = cum + off
        o1b = o1[ci * CH:(ci + 1) * CH, :]
        o2b = o2[ci * CH:(ci + 1) * CH, :]
        d1 = jnp.sum(pos * o1b, axis=1, keepdims=True)
        d2 = jnp.sum(pos * o2b, axis=1, keepdims=True)
        dest_ref[ci * CH:(ci + 1) * CH, 0:1] = d1.astype(jnp.int32)
        dest_ref[ci * CH:(ci + 1) * CH, 1:2] = d2.astype(jnp.int32)
        carry = carry + jnp.sum(blk, axis=0, keepdims=True)

    tiles = padded * (1.0 / TB)
    cumtiles = jnp.dot(tiles, (lr <= lc).astype(F32),
                       preferred_element_type=F32)
    jl = jax.lax.broadcasted_iota(jnp.int32, (1, 128), 1).astype(F32)
    te = jnp.zeros((1, 128), F32)
    for e in range(N_EXP - 1):
        ce = jnp.sum(jnp.where(lane[0:1, :] == e, cumtiles, 0.0))
        te = te + (jl >= ce).astype(F32)
    te_ref[...] = te.astype(jnp.int32)


def _router_call(xa, fw, gw_bf, gb):
    return pl.pallas_call(
        _router_body,
        out_shape=(
            jax.ShapeDtypeStruct((T, D_MODEL), F32),
            jax.ShapeDtypeStruct((T, 2), F32),
            jax.ShapeDtypeStruct((T, 2), jnp.int32),
            jax.ShapeDtypeStruct((1, 128), jnp.int32),
        ),
    )(xa, fw, gw_bf, gb)


_SC_CHUNK = 64
_SC_CORES = 2
_SC_SUBCORES = 16


def _make_sc_dispatch():
    nw = _SC_CORES * _SC_SUBCORES
    a_per_w = NA // nw
    mesh = plsc.VectorSubcoreMesh(core_axis_name="c", subcore_axis_name="s")

    @functools.partial(
        pl.kernel, mesh=mesh,
        out_type=jax.ShapeDtypeStruct((NP, D_MODEL), F32),
        scratch_types=[
            pltpu.VMEM((_SC_CHUNK,), jnp.int32),
            pltpu.VMEM((_SC_CHUNK,), jnp.int32),
            pltpu.VMEM((_SC_CHUNK, D_MODEL), F32),
            pltpu.SemaphoreType.DMA,
        ],
    )
    def dispatch(x2_hbm, tok_hbm, dest_hbm, xs_hbm, tok_v, dest_v, rows_v,
                 sem):
        wid = lax.axis_index("s") * _SC_CORES + lax.axis_index("c")
        for ci in range(a_per_w // _SC_CHUNK):
            base = wid * a_per_w + ci * _SC_CHUNK
            pltpu.sync_copy(tok_hbm.at[pl.ds(base, _SC_CHUNK)], tok_v)
            pltpu.sync_copy(dest_hbm.at[pl.ds(base, _SC_CHUNK)], dest_v)
            pltpu.async_copy(x2_hbm.at[tok_v], rows_v, sem).wait()
            pltpu.async_copy(rows_v, xs_hbm.at[dest_v], sem).wait()

    return dispatch


def _make_sc_combine():
    nw = _SC_CORES * _SC_SUBCORES
    a_per_w = NA // nw
    mesh = plsc.VectorSubcoreMesh(core_axis_name="c", subcore_axis_name="s")

    @functools.partial(
        pl.kernel, mesh=mesh,
        out_type=jax.ShapeDtypeStruct((NA, D_MODEL), F32),
        scratch_types=[
            pltpu.VMEM((_SC_CHUNK,), jnp.int32),
            pltpu.VMEM((_SC_CHUNK, D_MODEL), F32),
            pltpu.SemaphoreType.DMA,
        ],
    )
    def combine(ys_hbm, dest_hbm, yg_hbm, dest_v, rows_v, sem):
        wid = lax.axis_index("s") * _SC_CORES + lax.axis_index("c")
        for ci in range(a_per_w // _SC_CHUNK):
            base = wid * a_per_w + ci * _SC_CHUNK
            pltpu.sync_copy(dest_hbm.at[pl.ds(base, _SC_CHUNK)], dest_v)
            pltpu.async_copy(ys_hbm.at[dest_v], rows_v, sem).wait()
            pltpu.sync_copy(rows_v, yg_hbm.at[pl.ds(base, _SC_CHUNK)])

    return combine


def _ffn_body(te_ref, xs_ref, up_ref, dn_ref, ys_ref):
    xb = xs_ref[...].astype(BF16)
    h = jnp.dot(xb, up_ref[0].astype(BF16), preferred_element_type=F32)
    u = h[:, :D_FF]
    g = h[:, D_FF:]
    act = (jax.nn.silu(g) * u).astype(BF16)
    ys_ref[...] = jnp.dot(act, dn_ref[0].astype(BF16),
                          preferred_element_type=F32)


def _ffn_call(te, xs, up_bf, dn_bf):
    grid_spec = pltpu.PrefetchScalarGridSpec(
        num_scalar_prefetch=1,
        grid=(NT,),
        in_specs=[
            pl.BlockSpec((TB, D_MODEL), lambda i, te: (i, 0)),
            pl.BlockSpec((1, D_MODEL, 2 * D_FF), lambda i, te: (te[i], 0, 0)),
            pl.BlockSpec((1, D_FF, D_MODEL), lambda i, te: (te[i], 0, 0)),
        ],
        out_specs=pl.BlockSpec((TB, D_MODEL), lambda i, te: (i, 0)),
    )
    return pl.pallas_call(
        _ffn_body, grid_spec=grid_spec,
        out_shape=jax.ShapeDtypeStruct((NP, D_MODEL), F32),
    )(te, xs, up_bf, dn_bf)


TC2 = 512


def _combine_body(xa_ref, yg_ref, w_ref, out_ref):
    w = w_ref[...]
    out_ref[...] = (xa_ref[...] + w[:, 0:1] * yg_ref[:, 0, :]
                    + w[:, 1:2] * yg_ref[:, 1, :])


def _combine_call(xa, yg3, w):
    return pl.pallas_call(
        _combine_body,
        grid=(T // TC2,),
        in_specs=[
            pl.BlockSpec((TC2, D_MODEL), lambda i: (i, 0)),
            pl.BlockSpec((TC2, 2, D_MODEL), lambda i: (i, 0, 0)),
            pl.BlockSpec((TC2, 2), lambda i: (i, 0)),
        ],
        out_specs=pl.BlockSpec((TC2, D_MODEL), lambda i: (i, 0)),
        out_shape=jax.ShapeDtypeStruct((T, D_MODEL), F32),
    )(xa, yg3, w)


def _rope_tables():
    pos = jnp.arange(T, dtype=F32) / ROPE_SCALE
    idx = jnp.arange(0, DH, 2, dtype=F32)
    inv_freq = 1.0 / (ROPE_BASE ** (idx / DH))
    freqs = jnp.einsum('t,f->tf', pos, inv_freq)
    s = jnp.sin(freqs)
    c = jnp.cos(freqs)
    ci = jnp.stack([c, c], axis=-1).reshape(T, DH)
    si = jnp.stack([-s, s], axis=-1).reshape(T, DH)
    return si, ci


def kernel(x, attn_norm_w, ffn_norm_w, qkv_w, qkv_b, proj_w, proj_b,
           attn_sink, gate_w, gate_b, up_w, down_w):
    B = x.shape[0]
    x2d = x.reshape(T, D_MODEL)
    sin, cos = _rope_tables()

    q, k, v = _qkv_call(
        x2d, attn_norm_w.reshape(1, D_MODEL), qkv_w.astype(BF16),
        qkv_b.reshape(1, QKV_DIM), sin, cos)

    xa = _attn_call(q, k, v, attn_sink.reshape(N_HEADS, 1, 1), x2d,
                    proj_w.astype(BF16), proj_b.reshape(1, D_MODEL))

    gw_pad = jnp.zeros((D_MODEL, 128), F32).at[:, :N_EXP].set(gate_w)
    gb_pad = jnp.zeros((1, 128), F32).at[0, :N_EXP].set(gate_b)
    x2, w12, dest, te = _router_call(
        xa, ffn_norm_w.reshape(1, D_MODEL), gw_pad.astype(BF16), gb_pad)

    dest_flat = dest.reshape(NA)
    tok = jnp.arange(NA, dtype=jnp.int32) // 2

    xs = _make_sc_dispatch()(x2, tok, dest_flat)
    ys = _ffn_call(te.reshape(128)[:NT], xs, up_w, down_w)
    yg = _make_sc_combine()(ys, dest_flat)

    out = _combine_call(xa, yg.reshape(T, 2, D_MODEL), w12)
    return out.reshape(B, T, D_MODEL)
